# baseline (speedup 1.0000x reference)
import functools

import jax
import jax.numpy as jnp
from jax import lax
from jax.experimental import pallas as pl
from jax.experimental.pallas import tpu as pltpu
from jax.experimental.pallas import tpu_sc as plsc

_NEG = -1e30
_POS = 1e30


def _layer_norm(h, g, b, eps=1e-5):
    mu = jnp.mean(h, axis=-1, keepdims=True)
    var = jnp.mean((h - mu) ** 2, axis=-1, keepdims=True)
    return (h - mu) * jax.lax.rsqrt(var + eps) * g + b


def _gate_body(E, x_ref, wg_ref, bg_ref, idx4_ref, w4_ref):
    s = jax.lax.dot_general(
        wg_ref[...], x_ref[...], (((1,), (1,)), ((), ())),
        preferred_element_type=jnp.float32) + bg_ref[...]
    iota = jax.lax.broadcasted_iota(jnp.int32, s.shape, 0)
    m1 = jnp.max(s, axis=0, keepdims=True)
    i1 = jnp.min(jnp.where(s == m1, iota, E), axis=0, keepdims=True)
    s_m = jnp.where(iota == i1, _NEG, s)
    m2 = jnp.max(s_m, axis=0, keepdims=True)
    i2 = jnp.min(jnp.where(s_m == m2, iota, E), axis=0, keepdims=True)
    n1 = jnp.min(s, axis=0, keepdims=True)
    j1 = jnp.min(jnp.where(s == n1, iota, E), axis=0, keepdims=True)
    s_q = jnp.where(iota == j1, _POS, s)
    n2 = jnp.min(s_q, axis=0, keepdims=True)
    j2 = jnp.min(jnp.where(s_q == n2, iota, E), axis=0, keepdims=True)
    e2 = jnp.exp(m2 - m1)
    z = 1.0 + e2
    eb = jnp.exp(n1 - n2)
    zb = 1.0 + eb
    idx4_ref[...] = jnp.concatenate([i1, i2, j1, j2], axis=0)
    w4_ref[...] = jnp.concatenate([1.0 / z, e2 / z, eb / zb, 1.0 / zb], axis=0)


def _gate(xf, Wg, bg):
    T, D = xf.shape
    E = Wg.shape[0]
    return pl.pallas_call(
        functools.partial(_gate_body, E),
        in_specs=[
            pl.BlockSpec((T, D), lambda: (0, 0)),
            pl.BlockSpec((E, D), lambda: (0, 0)),
            pl.BlockSpec((E, 1), lambda: (0, 0)),
        ],
        out_specs=[
            pl.BlockSpec((4, T), lambda: (0, 0)),
            pl.BlockSpec((4, T), lambda: (0, 0)),
        ],
        out_shape=[
            jax.ShapeDtypeStruct((4, T), jnp.int32),
            jax.ShapeDtypeStruct((4, T), jnp.float32),
        ],
    )(xf, Wg, bg.reshape(E, 1))


def _route(idx4, T, E, BM, NB, P):
    a = idx4.T.reshape(-1)
    oh = (a[:, None] == jnp.arange(E, dtype=jnp.int32)[None, :]).astype(jnp.int32)
    csum = jnp.cumsum(oh, axis=0)
    pos = jnp.take_along_axis(csum, a[:, None], axis=1)[:, 0] - 1
    n_e = csum[-1]
    nb_e = (n_e + BM - 1) // BM
    ends = jnp.cumsum(nb_e)
    group_start = (ends - nb_e) * BM
    slot = group_start[a] + pos
    tok_flat = jnp.repeat(jnp.arange(T, dtype=jnp.int32), 4)
    tok_sorted = jnp.zeros((P,), jnp.int32).at[slot].set(tok_flat)
    block_expert = jnp.clip(
        jnp.searchsorted(ends, jnp.arange(NB, dtype=jnp.int32), side="right"),
        0, E - 1).astype(jnp.int32)
    slot4 = slot.reshape(T, 4).T.reshape(-1)
    return tok_sorted, block_expert, slot4


def _sc_gather(table, idx):
    n = idx.shape[0]
    D = table.shape[1]
    info = plsc.get_sparse_core_info()
    NC = info.num_cores
    NW = NC * info.num_subcores
    per_w = n // NW
    CH = 64
    while per_w % CH:
        CH //= 2
    iters = per_w // CH
    mesh = plsc.VectorSubcoreMesh(core_axis_name="c", subcore_axis_name="s")

    @functools.partial(
        pl.kernel, mesh=mesh,
        out_type=jax.ShapeDtypeStruct((n, D), table.dtype),
        scratch_types=[
            pltpu.VMEM((CH,), jnp.int32),
            pltpu.VMEM((CH, D), table.dtype),
            pltpu.SemaphoreType.DMA,
        ],
    )
    def k(table_hbm, idx_hbm, out_hbm, idx_v, rows_v, sem):
        wid = lax.axis_index("s") * NC + lax.axis_index("c")
        base = wid * per_w

        def body(i, carry):
            off = base + i * CH
            pltpu.sync_copy(idx_hbm.at[pl.ds(off, CH)], idx_v)
            pltpu.async_copy(table_hbm.at[idx_v], rows_v, sem).wait()
            pltpu.sync_copy(rows_v, out_hbm.at[pl.ds(off, CH)])
            return carry

        lax.fori_loop(0, iters, body, 0)

    return k(table, idx)


def _ffn_body(be_ref, xs_ref, w1_ref, b1_ref, g1_ref, be1_ref,
              w2_ref, b2_ref, g2_ref, be2_ref, ys_ref):
    xv = xs_ref[...]
    h = jax.lax.dot_general(
        xv, w1_ref[0], (((1,), (1,)), ((), ())),
        preferred_element_type=jnp.float32) + b1_ref[0]
    h = _layer_norm(h, g1_ref[0], be1_ref[0])
    h = jnp.maximum(h, 0.0)
    o = jax.lax.dot_general(
        h, w2_ref[0], (((1,), (1,)), ((), ())),
        preferred_element_type=jnp.float32) + b2_ref[0]
    ys_ref[...] = _layer_norm(o, g2_ref[0], be2_ref[0])


def _grouped_ffn(xs, block_expert, W1, b1, g1, be1, W2, b2, g2, be2, BM, NB):
    P, D = xs.shape
    E = W1.shape[0]

    def wmap(b, be_ref):
        return (be_ref[b], 0, 0)

    grid_spec = pltpu.PrefetchScalarGridSpec(
        num_scalar_prefetch=1,
        grid=(NB,),
        in_specs=[
            pl.BlockSpec((BM, D), lambda b, be_ref: (b, 0)),
            pl.BlockSpec((1, D, D), wmap),
            pl.BlockSpec((1, 1, D), wmap),
            pl.BlockSpec((1, 1, D), wmap),
            pl.BlockSpec((1, 1, D), wmap),
            pl.BlockSpec((1, D, D), wmap),
            pl.BlockSpec((1, 1, D), wmap),
            pl.BlockSpec((1, 1, D), wmap),
            pl.BlockSpec((1, 1, D), wmap),
        ],
        out_specs=pl.BlockSpec((BM, D), lambda b, be_ref: (b, 0)),
    )
    return pl.pallas_call(
        _ffn_body,
        grid_spec=grid_spec,
        out_shape=jax.ShapeDtypeStruct((P, D), jnp.float32),
    )(block_expert, xs,
      W1, b1.reshape(E, 1, D), g1.reshape(E, 1, D), be1.reshape(E, 1, D),
      W2, b2.reshape(E, 1, D), g2.reshape(E, 1, D), be2.reshape(E, 1, D))


def _epi_body(g_ref, w_ref, x_ref, out_ref, top_ref, bot_ref, ss_ref):
    w = w_ref[...]
    top = w[:, 0:1] * g_ref[0] + w[:, 1:2] * g_ref[1]
    bot = w[:, 2:3] * g_ref[2] + w[:, 3:4] * g_ref[3]
    out_ref[...] = top + x_ref[...]
    top_ref[...] = top
    bot_ref[...] = bot
    d = top - bot
    ss_ref[...] = jnp.full(ss_ref.shape, jnp.sum(d * d), jnp.float32)


def _epilogue(gath, w4t, xf, BTE):
    T, D = xf.shape
    nb = T // BTE
    return pl.pallas_call(
        _epi_body,
        grid=(nb,),
        in_specs=[
            pl.BlockSpec((4, BTE, D), lambda tb: (0, tb, 0)),
            pl.BlockSpec((BTE, 4), lambda tb: (tb, 0)),
            pl.BlockSpec((BTE, D), lambda tb: (tb, 0)),
        ],
        out_specs=[
            pl.BlockSpec((BTE, D), lambda tb: (tb, 0)),
            pl.BlockSpec((BTE, D), lambda tb: (tb, 0)),
            pl.BlockSpec((BTE, D), lambda tb: (tb, 0)),
            pl.BlockSpec((8, 128), lambda tb: (tb, 0)),
        ],
        out_shape=[
            jax.ShapeDtypeStruct((T, D), jnp.float32),
            jax.ShapeDtypeStruct((T, D), jnp.float32),
            jax.ShapeDtypeStruct((T, D), jnp.float32),
            jax.ShapeDtypeStruct((nb * 8, 128), jnp.float32),
        ],
    )(gath, w4t, xf)


def kernel(x, Wg, bg, W1, b1, g1, be1, W2, b2, g2, be2):
    B_, N_, D_ = x.shape
    T = B_ * N_
    E = Wg.shape[0]
    xf = x.reshape(T, D_)

    BM = 256
    NB = 4 * T // BM + E
    P = NB * BM

    idx4, w4 = _gate(xf, Wg, bg)
    tok_sorted, block_expert, slot4 = _route(idx4, T, E, BM, NB, P)
    xs = _sc_gather(xf, tok_sorted)
    ys = _grouped_ffn(xs, block_expert, W1, b1, g1, be1, W2, b2, g2, be2,
                      BM, NB)
    gath = _sc_gather(ys, slot4).reshape(4, T, D_)
    out, top, bot, ss = _epilogue(gath, w4.T, xf, BTE=min(512, T))
    total_ss = jnp.sum(ss[::8, 0])
    loss = jnp.mean(1.0 / (jnp.sqrt(total_ss) + 1e-8))
    return (out.reshape(B_, N_, D_),
            top.reshape(B_, N_, D_),
            bot.reshape(B_, N_, D_),
            loss)


# speedup vs baseline: 1.2636x; 1.2636x over previous
import functools

import jax
import jax.numpy as jnp
from jax import lax
from jax.experimental import pallas as pl
from jax.experimental.pallas import tpu as pltpu
from jax.experimental.pallas import tpu_sc as plsc

_NEG = -1e30
_POS = 1e30


def _layer_norm(h, g, b, eps=1e-5):
    mu = jnp.mean(h, axis=-1, keepdims=True)
    var = jnp.mean((h - mu) ** 2, axis=-1, keepdims=True)
    return (h - mu) * jax.lax.rsqrt(var + eps) * g + b


def _gate_body(E, x_ref, wg_ref, bg_ref, idx4_ref, w4_ref, xbf_ref):
    s = jax.lax.dot_general(
        wg_ref[...], x_ref[...], (((1,), (1,)), ((), ())),
        preferred_element_type=jnp.float32) + bg_ref[...]
    iota = jax.lax.broadcasted_iota(jnp.int32, s.shape, 0)
    m1 = jnp.max(s, axis=0, keepdims=True)
    i1 = jnp.min(jnp.where(s == m1, iota, E), axis=0, keepdims=True)
    s_m = jnp.where(iota == i1, _NEG, s)
    m2 = jnp.max(s_m, axis=0, keepdims=True)
    i2 = jnp.min(jnp.where(s_m == m2, iota, E), axis=0, keepdims=True)
    n1 = jnp.min(s, axis=0, keepdims=True)
    j1 = jnp.min(jnp.where(s == n1, iota, E), axis=0, keepdims=True)
    s_q = jnp.where(iota == j1, _POS, s)
    n2 = jnp.min(s_q, axis=0, keepdims=True)
    j2 = jnp.min(jnp.where(s_q == n2, iota, E), axis=0, keepdims=True)
    e2 = jnp.exp(m2 - m1)
    z = 1.0 + e2
    eb = jnp.exp(n1 - n2)
    zb = 1.0 + eb
    idx4_ref[...] = jnp.concatenate([i1, i2, j1, j2], axis=0)
    w4_ref[...] = jnp.concatenate([1.0 / z, e2 / z, eb / zb, 1.0 / zb], axis=0)
    xbf_ref[...] = x_ref[...].astype(jnp.bfloat16)


def _gate(xf, Wg, bg):
    T, D = xf.shape
    E = Wg.shape[0]
    return pl.pallas_call(
        functools.partial(_gate_body, E),
        in_specs=[
            pl.BlockSpec((T, D), lambda: (0, 0)),
            pl.BlockSpec((E, D), lambda: (0, 0)),
            pl.BlockSpec((E, 1), lambda: (0, 0)),
        ],
        out_specs=[
            pl.BlockSpec((4, T), lambda: (0, 0)),
            pl.BlockSpec((4, T), lambda: (0, 0)),
            pl.BlockSpec((T, D), lambda: (0, 0)),
        ],
        out_shape=[
            jax.ShapeDtypeStruct((4, T), jnp.int32),
            jax.ShapeDtypeStruct((4, T), jnp.float32),
            jax.ShapeDtypeStruct((T, D), jnp.bfloat16),
        ],
    )(xf, Wg, bg.reshape(E, 1))


def _route(idx4, T, E, BM, NB, P):
    a = idx4.T.reshape(-1)
    oh = (a[:, None] == jnp.arange(E, dtype=jnp.int32)[None, :]).astype(jnp.int32)
    csum = jnp.cumsum(oh, axis=0)
    pos = jnp.take_along_axis(csum, a[:, None], axis=1)[:, 0] - 1
    n_e = csum[-1]
    nb_e = (n_e + BM - 1) // BM
    ends = jnp.cumsum(nb_e)
    group_start = (ends - nb_e) * BM
    slot = group_start[a] + pos
    tok_flat = jnp.repeat(jnp.arange(T, dtype=jnp.int32), 4)
    tok_sorted = jnp.zeros((P,), jnp.int32).at[slot].set(tok_flat)
    block_expert = jnp.clip(
        jnp.searchsorted(ends, jnp.arange(NB, dtype=jnp.int32), side="right"),
        0, E - 1).astype(jnp.int32)
    slot4 = slot.reshape(T, 4).T.reshape(-1)
    return tok_sorted, block_expert, slot4


def _sc_gather(table, idx):
    n = idx.shape[0]
    D = table.shape[1]
    info = plsc.get_sparse_core_info()
    NC = info.num_cores
    NW = NC * info.num_subcores
    per_w = n // NW
    CH = 64
    while per_w % CH:
        CH //= 2
    iters = per_w // CH
    mesh = plsc.VectorSubcoreMesh(core_axis_name="c", subcore_axis_name="s")

    @functools.partial(
        pl.kernel, mesh=mesh,
        out_type=jax.ShapeDtypeStruct((n, D), table.dtype),
        scratch_types=[
            pltpu.VMEM((CH,), jnp.int32),
            pltpu.VMEM((CH, D), table.dtype),
            pltpu.SemaphoreType.DMA,
        ],
    )
    def k(table_hbm, idx_hbm, out_hbm, idx_v, rows_v, sem):
        wid = lax.axis_index("s") * NC + lax.axis_index("c")
        base = wid * per_w

        def body(i, carry):
            off = base + i * CH
            pltpu.sync_copy(idx_hbm.at[pl.ds(off, CH)], idx_v)
            pltpu.async_copy(table_hbm.at[idx_v], rows_v, sem).wait()
            pltpu.sync_copy(rows_v, out_hbm.at[pl.ds(off, CH)])
            return carry

        lax.fori_loop(0, iters, body, 0)

    return k(table, idx)


def _ffn_body(be_ref, tok_ref, xbf_ref, w1_ref, b1_ref, g1_ref, be1_ref,
              w2_ref, b2_ref, g2_ref, be2_ref, ys_ref):
    tids = tok_ref[0]
    iota_t = jax.lax.broadcasted_iota(jnp.int32, (xbf_ref.shape[0],
                                                  tids.shape[1]), 0)
    st = (iota_t == tids).astype(jnp.bfloat16)
    xv = jax.lax.dot_general(
        st, xbf_ref[...], (((0,), (0,)), ((), ())),
        preferred_element_type=jnp.float32)
    h = jax.lax.dot_general(
        xv, w1_ref[0], (((1,), (1,)), ((), ())),
        preferred_element_type=jnp.float32) + b1_ref[0]
    h = _layer_norm(h, g1_ref[0], be1_ref[0])
    h = jnp.maximum(h, 0.0)
    o = jax.lax.dot_general(
        h, w2_ref[0], (((1,), (1,)), ((), ())),
        preferred_element_type=jnp.float32) + b2_ref[0]
    ys_ref[...] = _layer_norm(o, g2_ref[0], be2_ref[0])


def _grouped_ffn(xbf, tok_sorted, block_expert,
                 W1, b1, g1, be1, W2, b2, g2, be2, BM, NB):
    T, D = xbf.shape
    E = W1.shape[0]
    P = NB * BM

    def wmap(b, be_ref):
        return (be_ref[b], 0, 0)

    grid_spec = pltpu.PrefetchScalarGridSpec(
        num_scalar_prefetch=1,
        grid=(NB,),
        in_specs=[
            pl.BlockSpec((1, 1, BM), lambda b, be_ref: (b, 0, 0)),
            pl.BlockSpec((T, D), lambda b, be_ref: (0, 0)),
            pl.BlockSpec((1, D, D), wmap),
            pl.BlockSpec((1, 1, D), wmap),
            pl.BlockSpec((1, 1, D), wmap),
            pl.BlockSpec((1, 1, D), wmap),
            pl.BlockSpec((1, D, D), wmap),
            pl.BlockSpec((1, 1, D), wmap),
            pl.BlockSpec((1, 1, D), wmap),
            pl.BlockSpec((1, 1, D), wmap),
        ],
        out_specs=pl.BlockSpec((BM, D), lambda b, be_ref: (b, 0)),
    )
    return pl.pallas_call(
        _ffn_body,
        grid_spec=grid_spec,
        out_shape=jax.ShapeDtypeStruct((P, D), jnp.float32),
    )(block_expert, tok_sorted.reshape(NB, 1, BM), xbf,
      W1, b1.reshape(E, 1, D), g1.reshape(E, 1, D), be1.reshape(E, 1, D),
      W2, b2.reshape(E, 1, D), g2.reshape(E, 1, D), be2.reshape(E, 1, D))


def _epi_body(g_ref, w_ref, x_ref, out_ref, top_ref, bot_ref, ss_ref):
    w = w_ref[...]
    top = w[:, 0:1] * g_ref[0] + w[:, 1:2] * g_ref[1]
    bot = w[:, 2:3] * g_ref[2] + w[:, 3:4] * g_ref[3]
    out_ref[...] = top + x_ref[...]
    top_ref[...] = top
    bot_ref[...] = bot
    d = top - bot
    ss_ref[...] = jnp.full(ss_ref.shape, jnp.sum(d * d), jnp.float32)


def _epilogue(gath, w4t, xf, BTE):
    T, D = xf.shape
    nb = T // BTE
    return pl.pallas_call(
        _epi_body,
        grid=(nb,),
        in_specs=[
            pl.BlockSpec((4, BTE, D), lambda tb: (0, tb, 0)),
            pl.BlockSpec((BTE, 4), lambda tb: (tb, 0)),
            pl.BlockSpec((BTE, D), lambda tb: (tb, 0)),
        ],
        out_specs=[
            pl.BlockSpec((BTE, D), lambda tb: (tb, 0)),
            pl.BlockSpec((BTE, D), lambda tb: (tb, 0)),
            pl.BlockSpec((BTE, D), lambda tb: (tb, 0)),
            pl.BlockSpec((8, 128), lambda tb: (tb, 0)),
        ],
        out_shape=[
            jax.ShapeDtypeStruct((T, D), jnp.float32),
            jax.ShapeDtypeStruct((T, D), jnp.float32),
            jax.ShapeDtypeStruct((T, D), jnp.float32),
            jax.ShapeDtypeStruct((nb * 8, 128), jnp.float32),
        ],
    )(gath, w4t, xf)


def kernel(x, Wg, bg, W1, b1, g1, be1, W2, b2, g2, be2):
    B_, N_, D_ = x.shape
    T = B_ * N_
    E = Wg.shape[0]
    xf = x.reshape(T, D_)

    BM = 256
    NB = 4 * T // BM + E
    P = NB * BM

    idx4, w4, xbf = _gate(xf, Wg, bg)
    tok_sorted, block_expert, slot4 = _route(idx4, T, E, BM, NB, P)
    ys = _grouped_ffn(xbf, tok_sorted, block_expert,
                      W1, b1, g1, be1, W2, b2, g2, be2, BM, NB)
    gath = _sc_gather(ys, slot4).reshape(4, T, D_)
    out, top, bot, ss = _epilogue(gath, w4.T, xf, BTE=min(512, T))
    total_ss = jnp.sum(ss[::8, 0])
    loss = jnp.mean(1.0 / (jnp.sqrt(total_ss) + 1e-8))
    return (out.reshape(B_, N_, D_),
            top.reshape(B_, N_, D_),
            bot.reshape(B_, N_, D_),
            loss)
